# baseline (speedup 1.0000x reference)
import functools

import jax
import jax.numpy as jnp
from jax import lax
from jax.experimental import pallas as pl
from jax.experimental.pallas import tpu as pltpu
from jax.experimental.pallas import tpu_sc as plsc

N = 10000
E = 320000
F_IN = 128
H = 64
T = 12
C = 10

G = 128
NSLICE = 16
SB = 160
EP = NSLICE * SB * G
NPAD = 10240
RANGE = 2560
NRANGE = 4
NOUT = NRANGE * RANGE
ACCR = RANGE + 128
RPT = ACCR // 16
OPT = RANGE // 16
CB = 6
CW = 128
BLK = 512
UNROLL = 8

_mesh = plsc.VectorSubcoreMesh(core_axis_name="c", subcore_axis_name="s")

_DNUMS16 = lax.GatherDimensionNumbers(
    offset_dims=(), collapsed_slice_dims=(0,), start_index_map=(0,))


def _take16(vec, lane):
    idx = jnp.full((16, 1), lane, jnp.int32)
    return lax.gather(vec, idx, _DNUMS16, (1,),
                      mode=lax.GatherScatterMode.PROMISE_IN_BOUNDS)


def _zero_fill_2d(ref, rows, cols):
    zero = jnp.zeros((16,), jnp.float32)

    def body(r, _):
        for j in range(cols // 16):
            ref[r, pl.ds(j * 16, 16)] = zero
        return 0

    lax.fori_loop(0, rows, body, 0)


def _localize_cols(col_t, lo):
    def body(b, _):
        for g in range(G // 16):
            cv = col_t[b, pl.ds(g * 16, 16)]
            m = (cv >= lo) & (cv < lo + RANGE)
            col_t[b, pl.ds(g * 16, 16)] = jnp.where(m, cv - lo, RANGE)
        return 0

    lax.fori_loop(0, SB, body, 0)


@functools.partial(
    pl.kernel,
    mesh=_mesh,
    compiler_params=pltpu.CompilerParams(use_tc_tiling_on_sc=True),
    out_type=[jax.ShapeDtypeStruct((NOUT, CW), jnp.float32) for _ in range(CB)],
    scratch_types=[
        pltpu.VMEM((SB, G), jnp.int32),
        pltpu.VMEM((SB, G), jnp.int32),
        pltpu.VMEM((SB, G), jnp.float32),
        pltpu.VMEM((G, CW), jnp.float32),
        pltpu.VMEM((G, CW), jnp.float32),
        pltpu.VMEM((G, CW), jnp.float32),
        pltpu.VMEM((G, CW), jnp.float32),
        pltpu.VMEM((8, CW), jnp.float32),
        pltpu.VMEM_SHARED((ACCR, CW), jnp.float32),
        pltpu.SemaphoreType.DMA,
        pltpu.SemaphoreType.DMA,
    ],
)
def _spmm_kernel(xw0, xw1, xw2, xw3, xw4, xw5, row_hbm, col_hbm, ew_hbm,
                 o0, o1, o2, o3, o4, o5,
                 row_t, col_t, ew_t, gb0, gb1, sb0, sb1, zbuf, acc_sh,
                 gsem0, gsem1):
    cc = lax.axis_index("c")
    s = lax.axis_index("s")
    gb = (gb0, gb1)
    sbv = (sb0, sb1)
    gsem = (gsem0, gsem1)
    ssem = (gsem0, gsem1)
    pltpu.sync_copy(row_hbm.at[s], row_t)
    pltpu.sync_copy(ew_hbm.at[s], ew_t)
    _zero_fill_2d(zbuf, 8, CW)

    for q in range(NRANGE // 2):
        lo = (cc * (NRANGE // 2) + q) * RANGE
        pltpu.sync_copy(col_hbm.at[s], col_t)
        _localize_cols(col_t, lo)
        for cb, (xw, out) in enumerate(zip((xw0, xw1, xw2, xw3, xw4, xw5),
                                           (o0, o1, o2, o3, o4, o5))):
            def zero_rows(z, _):
                pltpu.sync_copy(zbuf, acc_sh.at[pl.ds(s * RPT + z * 8, 8)])
                return 0

            lax.fori_loop(0, RPT // 8, zero_rows, 0)
            plsc.subcore_barrier()

            def scale_batch(b, src, dst):
                def scale(g, _):
                    wv = ew_t[b, pl.ds(g * 16, 16)]

                    def lane_body(lane, _):
                        e = g * 16 + lane
                        w = _take16(wv, lane)
                        for j in range(CW // 16):
                            dst[e, pl.ds(j * 16, 16)] = (
                                src[e, pl.ds(j * 16, 16)] * w)
                        return 0

                    lax.fori_loop(0, 16, lane_body, 0)
                    return 0

                lax.fori_loop(0, G // 16, scale, 0)

            def body(b, _):
                pltpu.async_copy(xw.at[row_t.at[b]], gb[0], gsem[0]).wait()
                scale_batch(b, gb[0], gb[0])
                pltpu.sync_copy(gb[0], acc_sh.at[col_t.at[b]], add=True)
                return 0

            lax.fori_loop(0, SB, body, 0)
            plsc.subcore_barrier()
            pltpu.sync_copy(
                acc_sh.at[pl.ds(s * OPT, OPT)],
                out.at[pl.ds(lo + s * OPT, OPT)],
            )
            plsc.subcore_barrier()


def _deg_body(colf_ref, ewf_ref, o_ref):
    base = pl.program_id(0) * BLK
    nodecol = (base + lax.broadcasted_iota(jnp.int32, (BLK, 1), 0)
               ).astype(jnp.float32)

    def body(r, acc):
        cv = colf_ref[r, :][None, :]
        wv = ewf_ref[r, :][None, :]
        return acc + jnp.where(nodecol == cv, wv, 0.0)

    acc = lax.fori_loop(0, E // G, body, jnp.zeros((BLK, G), jnp.float32))
    deg = jnp.sum(acc, axis=1, keepdims=True)
    o_ref[...] = jnp.broadcast_to(deg, (BLK, CW))


def _mm1_body(x2_ref, w_ref, deg_ref, o0, o1, o2, o3, o4, o5):
    dinv = lax.rsqrt(deg_ref[...][:, :1] + 1.0)
    y = jnp.dot(x2_ref[...], w_ref[...], preferred_element_type=jnp.float32)
    y = y * dinv
    for cb, o in enumerate((o0, o1, o2, o3, o4, o5)):
        o[...] = y[:, cb * CW:(cb + 1) * CW]


def _combine_body(s0, s1, s2, s3, s4, s5, x0, x1, x2, x3, x4, x5,
                  deg_ref, bt_ref, w_ref, o0, o1, o2, o3, o4, o5):
    dinv = lax.rsqrt(deg_ref[...][:, :1] + 1.0)
    bt = bt_ref[...]
    w = w_ref[...]
    y = None
    for cb, (sref, xref) in enumerate(zip((s0, s1, s2, s3, s4, s5),
                                          (x0, x1, x2, x3, x4, x5))):
        m = sref[...] + xref[...]
        h = jnp.maximum(m * dinv + bt[cb * CW:(cb + 1) * CW], 0.0)
        part = jnp.dot(h, w[cb * CW:(cb + 1) * CW, :],
                       preferred_element_type=jnp.float32)
        y = part if y is None else y + part
    y = y * dinv
    for cb, o in enumerate((o0, o1, o2, o3, o4, o5)):
        o[...] = y[:, cb * CW:(cb + 1) * CW]


def _head_body(s0, s1, s2, s3, s4, s5, x0, x1, x2, x3, x4, x5,
               deg_ref, bt_ref, wseq_ref, bseq_ref, wcls_ref, bcls_ref,
               o_ref):
    dinv = lax.rsqrt(deg_ref[...][:, :1] + 1.0)
    bt = bt_ref[...]
    wseq = wseq_ref[...]
    y = None
    for cb, (sref, xref) in enumerate(zip((s0, s1, s2, s3, s4, s5),
                                          (x0, x1, x2, x3, x4, x5))):
        m = sref[...] + xref[...]
        h = jnp.maximum(m * dinv + bt[cb * CW:(cb + 1) * CW], 0.0)
        part = jnp.dot(h, wseq[cb * CW:(cb + 1) * CW, :],
                       preferred_element_type=jnp.float32)
        y = part if y is None else y + part
    hs = jnp.maximum(y + bseq_ref[...], 0.0)
    o_ref[...] = (jnp.dot(hs, wcls_ref[...], preferred_element_type=jnp.float32)
                  + bcls_ref[...])


def _grid():
    return (pl.cdiv(N, BLK),)


def _row_spec(w):
    return pl.BlockSpec((BLK, w), lambda i: (i, 0))


def _full_spec(shape):
    nd = len(shape)
    return pl.BlockSpec(shape, lambda i: (0,) * nd)


def kernel(x, edge_index, edge_weight, W1, b1, W2, b2, W3, b3,
           W_seq, b_seq, W_cls, b_cls):
    f32 = jnp.float32
    x2 = jnp.transpose(x, (0, 2, 1)).reshape(N, T * F_IN)
    row = edge_index[0]
    col = edge_index[1]
    pad = EP - E
    rowp = jnp.concatenate([row, jnp.zeros((pad,), row.dtype)]).reshape(NSLICE, SB, G)
    colp = jnp.concatenate([col, jnp.zeros((pad,), col.dtype)]).reshape(NSLICE, SB, G)
    ewp = jnp.concatenate([edge_weight, jnp.zeros((pad,), f32)]).reshape(NSLICE, SB, G)
    eyeT = jnp.eye(T, dtype=f32)
    W1bd = jnp.kron(eyeT, W1)
    W2bd = jnp.kron(eyeT, W2)
    W3bd = jnp.kron(eyeT, W3)
    b1t = jnp.tile(b1, T)
    b2t = jnp.tile(b2, T)
    b3t = jnp.tile(b3, T)
    Wclsp = jnp.pad(W_cls, ((0, 0), (0, 128 - C)))
    bclsp = jnp.pad(b_cls, (0, 128 - C))

    colf = col.astype(f32).reshape(E // G, G)
    ewf = edge_weight.reshape(E // G, G)
    degk = pl.pallas_call(
        _deg_body,
        grid=_grid(),
        in_specs=[_full_spec((E // G, G)), _full_spec((E // G, G))],
        out_specs=_row_spec(CW),
        out_shape=jax.ShapeDtypeStruct((N, CW), f32),
    )
    deg = degk(colf, ewf)

    mm1 = pl.pallas_call(
        _mm1_body,
        grid=_grid(),
        in_specs=[_row_spec(T * F_IN), _full_spec((T * F_IN, T * H)),
                  _row_spec(CW)],
        out_specs=[_row_spec(CW)] * CB,
        out_shape=[jax.ShapeDtypeStruct((N, CW), f32)] * CB,
    )
    xw = mm1(x2, W1bd, deg)

    combine = pl.pallas_call(
        _combine_body,
        grid=_grid(),
        in_specs=[_row_spec(CW)] * (2 * CB)
        + [_row_spec(CW), _full_spec((T * H,)), _full_spec((T * H, T * H))],
        out_specs=[_row_spec(CW)] * CB,
        out_shape=[jax.ShapeDtypeStruct((N, CW), f32)] * CB,
    )

    sparts = _spmm_kernel(*xw, rowp, colp, ewp)
    xw = combine(*sparts, *xw, deg, b1t, W2bd)
    sparts = _spmm_kernel(*xw, rowp, colp, ewp)
    xw = combine(*sparts, *xw, deg, b2t, W3bd)
    sparts = _spmm_kernel(*xw, rowp, colp, ewp)

    head = pl.pallas_call(
        _head_body,
        grid=_grid(),
        in_specs=[_row_spec(CW)] * (2 * CB)
        + [_row_spec(CW), _full_spec((T * H,)), _full_spec((T * H, H)),
           _full_spec((H,)), _full_spec((H, 128)), _full_spec((128,))],
        out_specs=_row_spec(128),
        out_shape=jax.ShapeDtypeStruct((N, 128), f32),
    )
    logits = head(*sparts, *xw, deg, b3t, W_seq, b_seq, Wclsp, bclsp)
    return logits[:, :C]


# speedup vs baseline: 1.0858x; 1.0858x over previous
import functools

import jax
import jax.numpy as jnp
from jax import lax
from jax.experimental import pallas as pl
from jax.experimental.pallas import tpu as pltpu
from jax.experimental.pallas import tpu_sc as plsc

N = 10000
E = 320000
F_IN = 128
H = 64
T = 12
C = 10

G = 128
NSLICE = 16
SB = 160
EP = NSLICE * SB * G
NPAD = 10240
RANGE = 2560
NRANGE = 4
NOUT = NRANGE * RANGE
ACCR = RANGE + 128
RPT = ACCR // 16
OPT = RANGE // 16
CB = 6
CW = 128
BLK = 512
UNROLL = 8

_mesh = plsc.VectorSubcoreMesh(core_axis_name="c", subcore_axis_name="s")

_DNUMS16 = lax.GatherDimensionNumbers(
    offset_dims=(), collapsed_slice_dims=(0,), start_index_map=(0,))


def _take16(vec, lane):
    idx = jnp.full((16, 1), lane, jnp.int32)
    return lax.gather(vec, idx, _DNUMS16, (1,),
                      mode=lax.GatherScatterMode.PROMISE_IN_BOUNDS)


def _zero_fill_2d(ref, rows, cols):
    zero = jnp.zeros((16,), jnp.float32)

    def body(r, _):
        for j in range(cols // 16):
            ref[r, pl.ds(j * 16, 16)] = zero
        return 0

    lax.fori_loop(0, rows, body, 0)


def _localize_cols(col_t, lo):
    def body(b, _):
        for g in range(G // 16):
            cv = col_t[b, pl.ds(g * 16, 16)]
            m = (cv >= lo) & (cv < lo + RANGE)
            col_t[b, pl.ds(g * 16, 16)] = jnp.where(m, cv - lo, RANGE)
        return 0

    lax.fori_loop(0, SB, body, 0)


@functools.partial(
    pl.kernel,
    mesh=_mesh,
    compiler_params=pltpu.CompilerParams(use_tc_tiling_on_sc=True),
    out_type=jax.ShapeDtypeStruct((NOUT, CW), jnp.float32),
    scratch_types=[
        pltpu.VMEM((SB, G), jnp.int32),
        pltpu.VMEM((SB, G), jnp.float32),
        pltpu.VMEM((G, CW), jnp.float32),
        pltpu.VMEM((8, CW), jnp.float32),
        pltpu.VMEM_SHARED((ACCR, CW), jnp.float32),
    ],
)
def _deg_kernel(col_hbm, ew_hbm, out_hbm, col_t, ew_t, gbuf, zbuf, acc_sh):
    cc = lax.axis_index("c")
    s = lax.axis_index("s")
    pltpu.sync_copy(ew_hbm.at[s], ew_t)
    _zero_fill_2d(zbuf, 8, CW)
    for q in range(NRANGE // 2):
        lo = (cc * (NRANGE // 2) + q) * RANGE
        pltpu.sync_copy(col_hbm.at[s], col_t)
        _localize_cols(col_t, lo)

        def zero_rows(z, _):
            pltpu.sync_copy(zbuf, acc_sh.at[pl.ds(s * RPT + z * 8, 8)])
            return 0

        lax.fori_loop(0, RPT // 8, zero_rows, 0)
        plsc.subcore_barrier()

        def body(b, _):
            def fill(g, _):
                wv = ew_t[b, pl.ds(g * 16, 16)]

                def lane_body(lane, _):
                    e = g * 16 + lane
                    w = _take16(wv, lane)
                    for j in range(CW // 16):
                        gbuf[e, pl.ds(j * 16, 16)] = (
                            gbuf[e, pl.ds(j * 16, 16)] * 0.0 + w)
                    return 0

                lax.fori_loop(0, 16, lane_body, 0)
                return 0

            lax.fori_loop(0, G // 16, fill, 0)
            pltpu.sync_copy(gbuf, acc_sh.at[col_t.at[b]], add=True)
            return 0

        lax.fori_loop(0, SB, body, 0)
        plsc.subcore_barrier()
        pltpu.sync_copy(
            acc_sh.at[pl.ds(s * OPT, OPT)],
            out_hbm.at[pl.ds(lo + s * OPT, OPT)],
        )
        plsc.subcore_barrier()


@functools.partial(
    pl.kernel,
    mesh=_mesh,
    compiler_params=pltpu.CompilerParams(use_tc_tiling_on_sc=True),
    out_type=[jax.ShapeDtypeStruct((NOUT, CW), jnp.float32) for _ in range(CB)],
    scratch_types=[
        pltpu.VMEM((SB, G), jnp.int32),
        pltpu.VMEM((SB, G), jnp.int32),
        pltpu.VMEM((SB, G), jnp.float32),
        pltpu.VMEM((G, CW), jnp.float32),
        pltpu.VMEM((G, CW), jnp.float32),
        pltpu.VMEM((G, CW), jnp.float32),
        pltpu.VMEM((G, CW), jnp.float32),
        pltpu.VMEM((8, CW), jnp.float32),
        pltpu.VMEM_SHARED((ACCR, CW), jnp.float32),
        pltpu.SemaphoreType.DMA,
        pltpu.SemaphoreType.DMA,
    ],
)
def _spmm_kernel(xw0, xw1, xw2, xw3, xw4, xw5, row_hbm, col_hbm, ew_hbm,
                 o0, o1, o2, o3, o4, o5,
                 row_t, col_t, ew_t, gb0, gb1, sb0, sb1, zbuf, acc_sh,
                 gsem0, gsem1):
    cc = lax.axis_index("c")
    s = lax.axis_index("s")
    gb = (gb0, gb1)
    sbv = (sb0, sb1)
    gsem = (gsem0, gsem1)
    ssem = (gsem0, gsem1)
    pltpu.sync_copy(row_hbm.at[s], row_t)
    pltpu.sync_copy(ew_hbm.at[s], ew_t)
    _zero_fill_2d(zbuf, 8, CW)

    for q in range(NRANGE // 2):
        lo = (cc * (NRANGE // 2) + q) * RANGE
        pltpu.sync_copy(col_hbm.at[s], col_t)
        _localize_cols(col_t, lo)
        for cb, (xw, out) in enumerate(zip((xw0, xw1, xw2, xw3, xw4, xw5),
                                           (o0, o1, o2, o3, o4, o5))):
            def zero_rows(z, _):
                pltpu.sync_copy(zbuf, acc_sh.at[pl.ds(s * RPT + z * 8, 8)])
                return 0

            lax.fori_loop(0, RPT // 8, zero_rows, 0)
            plsc.subcore_barrier()

            def scale_batch(b, src, dst):
                def scale(g, _):
                    wv = ew_t[b, pl.ds(g * 16, 16)]

                    def lane_body(lane, _):
                        e = g * 16 + lane
                        w = _take16(wv, lane)
                        for j in range(CW // 16):
                            dst[e, pl.ds(j * 16, 16)] = (
                                src[e, pl.ds(j * 16, 16)] * w)
                        return 0

                    lax.fori_loop(0, 16, lane_body, 0)
                    return 0

                lax.fori_loop(0, G // 16, scale, 0)

            def body(b, _):
                pltpu.async_copy(xw.at[row_t.at[b]], gb[0], gsem[0]).wait()
                scale_batch(b, gb[0], gb[0])
                pltpu.sync_copy(gb[0], acc_sh.at[col_t.at[b]], add=True)
                return 0

            lax.fori_loop(0, SB, body, 0)
            plsc.subcore_barrier()
            pltpu.sync_copy(
                acc_sh.at[pl.ds(s * OPT, OPT)],
                out.at[pl.ds(lo + s * OPT, OPT)],
            )
            plsc.subcore_barrier()


def _mm1_body(x2_ref, w_ref, deg_ref, o0, o1, o2, o3, o4, o5):
    dinv = lax.rsqrt(deg_ref[...][:, :1] + 1.0)
    y = jnp.dot(x2_ref[...], w_ref[...], preferred_element_type=jnp.float32)
    y = y * dinv
    for cb, o in enumerate((o0, o1, o2, o3, o4, o5)):
        o[...] = y[:, cb * CW:(cb + 1) * CW]


def _combine_body(s0, s1, s2, s3, s4, s5, x0, x1, x2, x3, x4, x5,
                  deg_ref, bt_ref, w_ref, o0, o1, o2, o3, o4, o5):
    dinv = lax.rsqrt(deg_ref[...][:, :1] + 1.0)
    bt = bt_ref[...]
    w = w_ref[...]
    y = None
    for cb, (sref, xref) in enumerate(zip((s0, s1, s2, s3, s4, s5),
                                          (x0, x1, x2, x3, x4, x5))):
        m = sref[...] + xref[...]
        h = jnp.maximum(m * dinv + bt[cb * CW:(cb + 1) * CW], 0.0)
        part = jnp.dot(h, w[cb * CW:(cb + 1) * CW, :],
                       preferred_element_type=jnp.float32)
        y = part if y is None else y + part
    y = y * dinv
    for cb, o in enumerate((o0, o1, o2, o3, o4, o5)):
        o[...] = y[:, cb * CW:(cb + 1) * CW]


def _head_body(s0, s1, s2, s3, s4, s5, x0, x1, x2, x3, x4, x5,
               deg_ref, bt_ref, wseq_ref, bseq_ref, wcls_ref, bcls_ref,
               o_ref):
    dinv = lax.rsqrt(deg_ref[...][:, :1] + 1.0)
    bt = bt_ref[...]
    wseq = wseq_ref[...]
    y = None
    for cb, (sref, xref) in enumerate(zip((s0, s1, s2, s3, s4, s5),
                                          (x0, x1, x2, x3, x4, x5))):
        m = sref[...] + xref[...]
        h = jnp.maximum(m * dinv + bt[cb * CW:(cb + 1) * CW], 0.0)
        part = jnp.dot(h, wseq[cb * CW:(cb + 1) * CW, :],
                       preferred_element_type=jnp.float32)
        y = part if y is None else y + part
    hs = jnp.maximum(y + bseq_ref[...], 0.0)
    o_ref[...] = (jnp.dot(hs, wcls_ref[...], preferred_element_type=jnp.float32)
                  + bcls_ref[...])


def _grid():
    return (pl.cdiv(N, BLK),)


def _row_spec(w):
    return pl.BlockSpec((BLK, w), lambda i: (i, 0))


def _full_spec(shape):
    nd = len(shape)
    return pl.BlockSpec(shape, lambda i: (0,) * nd)


def kernel(x, edge_index, edge_weight, W1, b1, W2, b2, W3, b3,
           W_seq, b_seq, W_cls, b_cls):
    f32 = jnp.float32
    x2 = jnp.transpose(x, (0, 2, 1)).reshape(N, T * F_IN)
    row = edge_index[0]
    col = edge_index[1]
    pad = EP - E
    rowp = jnp.concatenate([row, jnp.zeros((pad,), row.dtype)]).reshape(NSLICE, SB, G)
    colp = jnp.concatenate([col, jnp.zeros((pad,), col.dtype)]).reshape(NSLICE, SB, G)
    ewp = jnp.concatenate([edge_weight, jnp.zeros((pad,), f32)]).reshape(NSLICE, SB, G)
    eyeT = jnp.eye(T, dtype=f32)
    W1bd = jnp.kron(eyeT, W1)
    W2bd = jnp.kron(eyeT, W2)
    W3bd = jnp.kron(eyeT, W3)
    b1t = jnp.tile(b1, T)
    b2t = jnp.tile(b2, T)
    b3t = jnp.tile(b3, T)
    Wclsp = jnp.pad(W_cls, ((0, 0), (0, 128 - C)))
    bclsp = jnp.pad(b_cls, (0, 128 - C))

    deg = _deg_kernel(colp, ewp)

    mm1 = pl.pallas_call(
        _mm1_body,
        grid=_grid(),
        in_specs=[_row_spec(T * F_IN), _full_spec((T * F_IN, T * H)),
                  _row_spec(CW)],
        out_specs=[_row_spec(CW)] * CB,
        out_shape=[jax.ShapeDtypeStruct((N, CW), f32)] * CB,
    )
    xw = mm1(x2, W1bd, deg)

    combine = pl.pallas_call(
        _combine_body,
        grid=_grid(),
        in_specs=[_row_spec(CW)] * (2 * CB)
        + [_row_spec(CW), _full_spec((T * H,)), _full_spec((T * H, T * H))],
        out_specs=[_row_spec(CW)] * CB,
        out_shape=[jax.ShapeDtypeStruct((N, CW), f32)] * CB,
    )

    sparts = _spmm_kernel(*xw, rowp, colp, ewp)
    xw = combine(*sparts, *xw, deg, b1t, W2bd)
    sparts = _spmm_kernel(*xw, rowp, colp, ewp)
    xw = combine(*sparts, *xw, deg, b2t, W3bd)
    sparts = _spmm_kernel(*xw, rowp, colp, ewp)

    head = pl.pallas_call(
        _head_body,
        grid=_grid(),
        in_specs=[_row_spec(CW)] * (2 * CB)
        + [_row_spec(CW), _full_spec((T * H,)), _full_spec((T * H, H)),
           _full_spec((H,)), _full_spec((H, 128)), _full_spec((128,))],
        out_specs=_row_spec(128),
        out_shape=jax.ShapeDtypeStruct((N, 128), f32),
    )
    logits = head(*sparts, *xw, deg, b3t, W_seq, b_seq, Wclsp, bclsp)
    return logits[:, :C]


# speedup vs baseline: 1.0893x; 1.0033x over previous
import functools

import jax
import jax.numpy as jnp
from jax import lax
from jax.experimental import pallas as pl
from jax.experimental.pallas import tpu as pltpu
from jax.experimental.pallas import tpu_sc as plsc

N = 10000
E = 320000
F_IN = 128
H = 64
T = 12
C = 10

G = 128
NSLICE = 16
SB = 160
EP = NSLICE * SB * G
NPAD = 10240
RANGE = 2560
NRANGE = 4
NOUT = NRANGE * RANGE
ACCR = RANGE + 128
RPT = ACCR // 16
OPT = RANGE // 16
CB = 6
CW = 128
BLK = 512
UNROLL = 8

_mesh = plsc.VectorSubcoreMesh(core_axis_name="c", subcore_axis_name="s")

_DNUMS16 = lax.GatherDimensionNumbers(
    offset_dims=(), collapsed_slice_dims=(0,), start_index_map=(0,))


def _take16(vec, lane):
    idx = jnp.full((16, 1), lane, jnp.int32)
    return lax.gather(vec, idx, _DNUMS16, (1,),
                      mode=lax.GatherScatterMode.PROMISE_IN_BOUNDS)


def _zero_fill_2d(ref, rows, cols):
    zero = jnp.zeros((16,), jnp.float32)

    def body(r, _):
        for j in range(cols // 16):
            ref[r, pl.ds(j * 16, 16)] = zero
        return 0

    lax.fori_loop(0, rows, body, 0)


def _localize_cols(col_t, lo):
    def body(b, _):
        for g in range(G // 16):
            cv = col_t[b, pl.ds(g * 16, 16)]
            m = (cv >= lo) & (cv < lo + RANGE)
            col_t[b, pl.ds(g * 16, 16)] = jnp.where(m, cv - lo, RANGE)
        return 0

    lax.fori_loop(0, SB, body, 0)


@functools.partial(
    pl.kernel,
    mesh=_mesh,
    compiler_params=pltpu.CompilerParams(use_tc_tiling_on_sc=True),
    out_type=jax.ShapeDtypeStruct((NOUT, CW), jnp.float32),
    scratch_types=[
        pltpu.VMEM((SB, G), jnp.int32),
        pltpu.VMEM((SB, G), jnp.float32),
        pltpu.VMEM((G, CW), jnp.float32),
        pltpu.VMEM((8, CW), jnp.float32),
        pltpu.VMEM_SHARED((ACCR, CW), jnp.float32),
    ],
)
def _deg_kernel(col_hbm, ew_hbm, out_hbm, col_t, ew_t, gbuf, zbuf, acc_sh):
    cc = lax.axis_index("c")
    s = lax.axis_index("s")
    pltpu.sync_copy(ew_hbm.at[s], ew_t)
    _zero_fill_2d(zbuf, 8, CW)
    for q in range(NRANGE // 2):
        lo = (cc * (NRANGE // 2) + q) * RANGE
        pltpu.sync_copy(col_hbm.at[s], col_t)
        _localize_cols(col_t, lo)

        def zero_rows(z, _):
            pltpu.sync_copy(zbuf, acc_sh.at[pl.ds(s * RPT + z * 8, 8)])
            return 0

        lax.fori_loop(0, RPT // 8, zero_rows, 0)
        plsc.subcore_barrier()

        def body(b, _):
            def fill(g, _):
                wv = ew_t[b, pl.ds(g * 16, 16)]

                def lane_body(lane, _):
                    e = g * 16 + lane
                    w = _take16(wv, lane)
                    for j in range(CW // 16):
                        gbuf[e, pl.ds(j * 16, 16)] = (
                            gbuf[e, pl.ds(j * 16, 16)] * 0.0 + w)
                    return 0

                lax.fori_loop(0, 16, lane_body, 0)
                return 0

            lax.fori_loop(0, G // 16, fill, 0)
            pltpu.sync_copy(gbuf, acc_sh.at[col_t.at[b]], add=True)
            return 0

        lax.fori_loop(0, SB, body, 0)
        plsc.subcore_barrier()
        pltpu.sync_copy(
            acc_sh.at[pl.ds(s * OPT, OPT)],
            out_hbm.at[pl.ds(lo + s * OPT, OPT)],
        )
        plsc.subcore_barrier()


@functools.partial(
    pl.kernel,
    mesh=_mesh,
    compiler_params=pltpu.CompilerParams(use_tc_tiling_on_sc=True),
    out_type=[jax.ShapeDtypeStruct((NOUT, CW), jnp.float32) for _ in range(CB)],
    scratch_types=[
        pltpu.VMEM((SB, G), jnp.int32),
        pltpu.VMEM((SB, G), jnp.int32),
        pltpu.VMEM((SB, G), jnp.float32),
        pltpu.VMEM((G, CW), jnp.float32),
        pltpu.VMEM((G, CW), jnp.float32),
        pltpu.VMEM((G, CW), jnp.float32),
        pltpu.VMEM((G, CW), jnp.float32),
        pltpu.VMEM((8, CW), jnp.float32),
        pltpu.VMEM_SHARED((ACCR, CW), jnp.float32),
        pltpu.SemaphoreType.DMA,
        pltpu.SemaphoreType.DMA,
        pltpu.SemaphoreType.DMA,
        pltpu.SemaphoreType.DMA,
    ],
)
def _spmm_kernel(xw0, xw1, xw2, xw3, xw4, xw5, row_hbm, col_hbm, ew_hbm,
                 o0, o1, o2, o3, o4, o5,
                 row_t, col_t, ew_t, gb0, gb1, gb2, gb3, zbuf, acc_sh,
                 sem0, sem1, sem2, sem3):
    cc = lax.axis_index("c")
    s = lax.axis_index("s")
    gb = (gb0, gb1, gb2, gb3)
    sem = (sem0, sem1, sem2, sem3)
    pltpu.sync_copy(row_hbm.at[s], row_t)
    pltpu.sync_copy(ew_hbm.at[s], ew_t)
    _zero_fill_2d(zbuf, 8, CW)

    for q in range(NRANGE // 2):
        lo = (cc * (NRANGE // 2) + q) * RANGE
        pltpu.sync_copy(col_hbm.at[s], col_t)
        _localize_cols(col_t, lo)
        for cb, (xw, out) in enumerate(zip((xw0, xw1, xw2, xw3, xw4, xw5),
                                           (o0, o1, o2, o3, o4, o5))):
            def zero_rows(z, _):
                pltpu.sync_copy(zbuf, acc_sh.at[pl.ds(s * RPT + z * 8, 8)])
                return 0

            lax.fori_loop(0, RPT // 8, zero_rows, 0)
            plsc.subcore_barrier()

            def scale_batch(b, src, dst):
                def scale(g, _):
                    wv = ew_t[b, pl.ds(g * 16, 16)]

                    def lane_body(lane, _):
                        e = g * 16 + lane
                        w = _take16(wv, lane)
                        for j in range(CW // 16):
                            dst[e, pl.ds(j * 16, 16)] = (
                                src[e, pl.ds(j * 16, 16)] * w)
                        return 0

                    lax.fori_loop(0, 16, lane_body, 0)
                    return 0

                lax.fori_loop(0, G // 16, scale, 0)

            def group(kk, _):
                base = kk * 2
                h0 = pltpu.async_copy(xw.at[row_t.at[base]], gb[0], sem[0])
                h1 = pltpu.async_copy(xw.at[row_t.at[base + 1]], gb[1], sem[1])
                h0.wait()
                scale_batch(base, gb[0], gb[0])
                pltpu.sync_copy(gb[0], acc_sh.at[col_t.at[base]], add=True)
                h1.wait()
                scale_batch(base + 1, gb[1], gb[1])
                pltpu.sync_copy(gb[1], acc_sh.at[col_t.at[base + 1]], add=True)
                return 0

            lax.fori_loop(0, SB // 2, group, 0)
            plsc.subcore_barrier()
            pltpu.sync_copy(
                acc_sh.at[pl.ds(s * OPT, OPT)],
                out.at[pl.ds(lo + s * OPT, OPT)],
            )
            plsc.subcore_barrier()


def _mm1_body(x2_ref, w_ref, deg_ref, o0, o1, o2, o3, o4, o5):
    dinv = lax.rsqrt(deg_ref[...][:, :1] + 1.0)
    y = jnp.dot(x2_ref[...], w_ref[...], preferred_element_type=jnp.float32)
    y = y * dinv
    for cb, o in enumerate((o0, o1, o2, o3, o4, o5)):
        o[...] = y[:, cb * CW:(cb + 1) * CW]


def _combine_body(s0, s1, s2, s3, s4, s5, x0, x1, x2, x3, x4, x5,
                  deg_ref, bt_ref, w_ref, o0, o1, o2, o3, o4, o5):
    dinv = lax.rsqrt(deg_ref[...][:, :1] + 1.0)
    bt = bt_ref[...]
    w = w_ref[...]
    y = None
    for cb, (sref, xref) in enumerate(zip((s0, s1, s2, s3, s4, s5),
                                          (x0, x1, x2, x3, x4, x5))):
        m = sref[...] + xref[...]
        h = jnp.maximum(m * dinv + bt[cb * CW:(cb + 1) * CW], 0.0)
        part = jnp.dot(h, w[cb * CW:(cb + 1) * CW, :],
                       preferred_element_type=jnp.float32)
        y = part if y is None else y + part
    y = y * dinv
    for cb, o in enumerate((o0, o1, o2, o3, o4, o5)):
        o[...] = y[:, cb * CW:(cb + 1) * CW]


def _head_body(s0, s1, s2, s3, s4, s5, x0, x1, x2, x3, x4, x5,
               deg_ref, bt_ref, wseq_ref, bseq_ref, wcls_ref, bcls_ref,
               o_ref):
    dinv = lax.rsqrt(deg_ref[...][:, :1] + 1.0)
    bt = bt_ref[...]
    wseq = wseq_ref[...]
    y = None
    for cb, (sref, xref) in enumerate(zip((s0, s1, s2, s3, s4, s5),
                                          (x0, x1, x2, x3, x4, x5))):
        m = sref[...] + xref[...]
        h = jnp.maximum(m * dinv + bt[cb * CW:(cb + 1) * CW], 0.0)
        part = jnp.dot(h, wseq[cb * CW:(cb + 1) * CW, :],
                       preferred_element_type=jnp.float32)
        y = part if y is None else y + part
    hs = jnp.maximum(y + bseq_ref[...], 0.0)
    o_ref[...] = (jnp.dot(hs, wcls_ref[...], preferred_element_type=jnp.float32)
                  + bcls_ref[...])


def _grid():
    return (pl.cdiv(N, BLK),)


def _row_spec(w):
    return pl.BlockSpec((BLK, w), lambda i: (i, 0))


def _full_spec(shape):
    nd = len(shape)
    return pl.BlockSpec(shape, lambda i: (0,) * nd)


def kernel(x, edge_index, edge_weight, W1, b1, W2, b2, W3, b3,
           W_seq, b_seq, W_cls, b_cls):
    f32 = jnp.float32
    x2 = jnp.transpose(x, (0, 2, 1)).reshape(N, T * F_IN)
    row = edge_index[0]
    col = edge_index[1]
    pad = EP - E
    rowp = jnp.concatenate([row, jnp.zeros((pad,), row.dtype)]).reshape(NSLICE, SB, G)
    colp = jnp.concatenate([col, jnp.zeros((pad,), col.dtype)]).reshape(NSLICE, SB, G)
    ewp = jnp.concatenate([edge_weight, jnp.zeros((pad,), f32)]).reshape(NSLICE, SB, G)
    eyeT = jnp.eye(T, dtype=f32)
    W1bd = jnp.kron(eyeT, W1)
    W2bd = jnp.kron(eyeT, W2)
    W3bd = jnp.kron(eyeT, W3)
    b1t = jnp.tile(b1, T)
    b2t = jnp.tile(b2, T)
    b3t = jnp.tile(b3, T)
    Wclsp = jnp.pad(W_cls, ((0, 0), (0, 128 - C)))
    bclsp = jnp.pad(b_cls, (0, 128 - C))

    deg = _deg_kernel(colp, ewp)

    mm1 = pl.pallas_call(
        _mm1_body,
        grid=_grid(),
        in_specs=[_row_spec(T * F_IN), _full_spec((T * F_IN, T * H)),
                  _row_spec(CW)],
        out_specs=[_row_spec(CW)] * CB,
        out_shape=[jax.ShapeDtypeStruct((N, CW), f32)] * CB,
    )
    xw = mm1(x2, W1bd, deg)

    combine = pl.pallas_call(
        _combine_body,
        grid=_grid(),
        in_specs=[_row_spec(CW)] * (2 * CB)
        + [_row_spec(CW), _full_spec((T * H,)), _full_spec((T * H, T * H))],
        out_specs=[_row_spec(CW)] * CB,
        out_shape=[jax.ShapeDtypeStruct((N, CW), f32)] * CB,
    )

    sparts = _spmm_kernel(*xw, rowp, colp, ewp)
    xw = combine(*sparts, *xw, deg, b1t, W2bd)
    sparts = _spmm_kernel(*xw, rowp, colp, ewp)
    xw = combine(*sparts, *xw, deg, b2t, W3bd)
    sparts = _spmm_kernel(*xw, rowp, colp, ewp)

    head = pl.pallas_call(
        _head_body,
        grid=_grid(),
        in_specs=[_row_spec(CW)] * (2 * CB)
        + [_row_spec(CW), _full_spec((T * H,)), _full_spec((T * H, H)),
           _full_spec((H,)), _full_spec((H, 128)), _full_spec((128,))],
        out_specs=_row_spec(128),
        out_shape=jax.ShapeDtypeStruct((N, 128), f32),
    )
    logits = head(*sparts, *xw, deg, b3t, W_seq, b_seq, Wclsp, bclsp)
    return logits[:, :C]
